# Initial kernel scaffold; baseline (speedup 1.0000x reference)
import jax
import jax.numpy as jnp
from jax.experimental import pallas as pl


def kernel(images, positions, t_val):
    raise NotImplementedError("write your pallas kernel here")


# speedup vs baseline: 18.4003x; 18.4003x over previous
import functools

import jax
import jax.numpy as jnp
import numpy as np
from jax import lax
from jax.experimental import pallas as pl
from jax.experimental.pallas import tpu as pltpu
from jax.experimental.pallas import tpu_sc as plsc

IMG = 72
CANVAS = 128
BATCH = 256
NSPRITE = 3
ROW_W = CANVAS * 3
CANVAS_W = CANVAS * ROW_W
SPRITE_W = IMG * IMG * 4
IMG_W = NSPRITE * SPRITE_W
NWORKERS = 32
SAMPLES_PER = BATCH // NWORKERS

def _sc_body(img_hbm, base_hbm, thr_hbm, tfill_hbm, out_hbm,
             img_v, canvas_v, pos_v, misc_v, dma_sem):
    c = lax.axis_index("c")
    s = lax.axis_index("s")
    wid = s * 2 + c

    pltpu.sync_copy(thr_hbm, misc_v.at[pl.ds(0, 16)])
    pltpu.sync_copy(tfill_hbm, misc_v.at[pl.ds(16, 16)])
    thr = misc_v[pl.ds(0, 16)]
    tfill = misc_v[pl.ds(16, 16)]
    iota = lax.iota(jnp.int32, 16)

    @pl.loop(0, SAMPLES_PER)
    def _sample(j):
        i = wid * SAMPLES_PER + j
        img_cp = pltpu.async_copy(img_hbm.at[i], img_v, dma_sem)
        pltpu.sync_copy(base_hbm.at[i], pos_v)

        @pl.loop(0, CANVAS_W // (16 * 16))
        def _fill(r):
            base = r * 256
            for u in range(16):
                canvas_v[pl.ds(base + u * 16, 16)] = tfill

        img_cp.wait()

        for p in (2, 1, 0):
            bvec = pos_v[pl.ds(p * 16, 16)]
            sbase = p * SPRITE_W
            for k in range(14):
                lanes = iota + (16 * k)
                valid = lanes < 216
                lv = jnp.where(valid, lanes, 0)
                pix = lv // 3
                ch = lv - pix * 3
                atab = pix * 4 + 3 + sbase
                rtab = pix * 4 + ch + sbase
                dtab = lv

                @pl.loop(0, IMG)
                def _row(sx, _bvec=bvec, _atab=atab, _rtab=rtab,
                         _dtab=dtab, _valid=valid, _k=k):
                    abase = sx * (IMG * 4)
                    alpha = plsc.load_gather(img_v, [_atab + abase])
                    rgb = plsc.load_gather(img_v, [_rtab + abase])
                    m = alpha > thr
                    if _k == 13:
                        m = m & _valid
                    d = _bvec + (_dtab + sx * ROW_W)
                    plsc.store_scatter(canvas_v, [d], rgb, mask=m)

        pltpu.sync_copy(canvas_v, out_hbm.at[i])


@jax.jit
def _paste(images2d, base48, thr16, tfill16):
    mesh = plsc.VectorSubcoreMesh(core_axis_name="c", subcore_axis_name="s")
    run = pl.kernel(
        _sc_body,
        out_type=jax.ShapeDtypeStruct((BATCH, CANVAS_W), jnp.float32),
        mesh=mesh,
        compiler_params=pltpu.CompilerParams(needs_layout_passes=False),
        scratch_types=[
            pltpu.VMEM((IMG_W,), jnp.float32),
            pltpu.VMEM((CANVAS_W,), jnp.float32),
            pltpu.VMEM((48,), jnp.int32),
            pltpu.VMEM((32,), jnp.float32),
            pltpu.SemaphoreType.DMA,
        ],
    )
    return run(images2d, base48, thr16, tfill16)


def kernel(images, positions, t_val=255.0):
    images = images.astype(jnp.float32)
    t_val = jnp.float32(t_val)
    positions = positions.reshape(BATCH, NSPRITE, 2)
    images2d = images.reshape(BATCH, IMG_W)
    base = positions[:, :, 1] * ROW_W + positions[:, :, 0] * 3
    base48 = jnp.broadcast_to(base[:, :, None], (BATCH, NSPRITE, 16))
    base48 = base48.reshape(BATCH, NSPRITE * 16).astype(jnp.int32)
    thr16 = jnp.where(t_val * 3 == 765.0, 0.0, jnp.inf)
    thr16 = jnp.full((16,), thr16, dtype=jnp.float32)
    tfill16 = jnp.full((16,), t_val, dtype=jnp.float32)
    out2d = _paste(images2d, base48, thr16, tfill16)
    return out2d.reshape(BATCH, CANVAS, CANVAS, 3)
